# baseline (speedup 1.0000x reference)
import jax
import jax.numpy as jnp
from jax import lax
from jax.experimental import pallas as pl
from jax.experimental.pallas import tpu as pltpu
from jax.experimental.pallas import tpu_sc as plsc

FRAME = 80
PMAX = 300
BATCH = 128
NF = 1000
NS = FRAME * NF
LANES = 16
NFP = 1024
NWORKERS = 32
RPW = BATCH // NWORKERS

HALO = 384
F0 = (0, 256, 512, 768)
NT = (256, 256, 256, 232)
NG = (16, 16, 16, 15)
SRC = (0, 80 * 256 - HALO, 80 * 512 - HALO, 80 * 768 - HALO)
LEN = (80 * 256, 80 * 512 - SRC[1], 80 * 768 - SRC[2], NS - SRC[3])
DST = (HALO, 0, 0, 0)
XBUF = HALO + LEN[1]

FSTRIDE = FRAME + 1
FROW = 81024
FSRC = tuple(FSTRIDE * f for f in F0)
FLEN = (20736, 20736, 20736, 18816)
FBUF = FSTRIDE * 256


def _rsqrt(v):
    i = plsc.bitcast(v, jnp.int32)
    i = 0x5F3759DF - lax.shift_right_arithmetic(i, 1)
    y = plsc.bitcast(i, jnp.float32)
    for _ in range(4):
        y = y * (1.5 - 0.5 * v * y * y)
    return y


def _sc_body(x_hbm, xs_hbm, per_hbm, out_hbm,
             xb0, xb1, fb0, fb1, pv, out_v, sem0, sem1, fsem0, fsem1):
    cid = lax.axis_index("c")
    sid = lax.axis_index("s")
    wid = sid * 2 + cid
    iota = lax.iota(jnp.int32, LANES)
    xbufs = (xb0, xb1)
    fbufs = (fb0, fb1)
    sems = (sem0, sem1)
    fsems = (fsem0, fsem1)
    zero = jnp.zeros((LANES,), jnp.float32)

    pltpu.sync_copy(per_hbm.at[pl.ds(wid * RPW * NFP, RPW * NFP)], pv)

    def lag_refs(q, b):
        return (x_hbm.at[b, 0, pl.ds(SRC[q], LEN[q])],
                xbufs[q % 2].at[pl.ds(DST[q], LEN[q])])

    def frame_refs(q, b):
        return (xs_hbm.at[b, pl.ds(FSRC[q], FLEN[q])],
                fbufs[q % 2].at[pl.ds(0, FLEN[q])])

    def start(q, b):
        s, d = lag_refs(q, b)
        pltpu.async_copy(s, d, sems[q % 2])
        s, d = frame_refs(q, b)
        pltpu.async_copy(s, d, fsems[q % 2])

    def wait(q, b):
        s, d = lag_refs(q, b)
        pltpu.make_async_copy(s, d, sems[q % 2]).wait()
        s, d = frame_refs(q, b)
        pltpu.make_async_copy(s, d, fsems[q % 2]).wait()

    def compute(q, r, b):
        buf = xbufs[q % 2]
        fbuf = fbufs[q % 2]
        if q == 0:
            for z in range(HALO // LANES):
                buf[pl.ds(z * LANES, LANES)] = zero

        pbase = r * NFP + F0[q]

        def group(g, carry):
            t = g * LANES + iota
            if q == 3:
                t = jnp.minimum(t, NT[3] - 1)
            p = plsc.load_gather(pv, [pbase + t])
            fb = t * FSTRIDE
            lb = t * FRAME + HALO - p
            acc = (zero,) * 12 + (fb, lb)

            def body4(k, acc):
                d0, d1, d2, d3, e0, e1, e2, e3, l0, l1, l2, l3, bi, li = acc
                fa = plsc.load_gather(fbuf, [bi])
                la = plsc.load_gather(buf, [li])
                fb_ = plsc.load_gather(fbuf, [bi + 1])
                lb_ = plsc.load_gather(buf, [li + 1])
                fc = plsc.load_gather(fbuf, [bi + 2])
                lc = plsc.load_gather(buf, [li + 2])
                fd = plsc.load_gather(fbuf, [bi + 3])
                ld = plsc.load_gather(buf, [li + 3])
                return (d0 + fa * la, d1 + fb_ * lb_, d2 + fc * lc, d3 + fd * ld,
                        e0 + fa * fa, e1 + fb_ * fb_, e2 + fc * fc, e3 + fd * fd,
                        l0 + la * la, l1 + lb_ * lb_, l2 + lc * lc, l3 + ld * ld,
                        bi + 4, li + 4)

            acc = lax.fori_loop(0, FRAME // 4, body4, acc, unroll=5)
            d = (acc[0] + acc[1]) + (acc[2] + acc[3])
            fe = (acc[4] + acc[5]) + (acc[6] + acc[7])
            le = (acc[8] + acc[9]) + (acc[10] + acc[11])
            res = d * _rsqrt(fe * le + 1e-9)
            out_v[pl.ds(pbase + g * LANES, LANES)] = res
            return carry

        lax.fori_loop(0, NG[q], group, 0)

    def row(r, carry):
        b = wid * RPW + r
        bnext = wid * RPW + jnp.minimum(r + 1, RPW - 1)
        start(1, b)
        wait(0, b)
        compute(0, r, b)
        start(2, b)
        wait(1, b)
        compute(1, r, b)
        start(3, b)
        wait(2, b)
        compute(2, r, b)
        start(0, bnext)
        wait(3, b)
        compute(3, r, b)
        return carry

    start(0, wid * RPW)
    lax.fori_loop(0, RPW, row, 0)
    wait(0, wid * RPW + RPW - 1)

    pltpu.sync_copy(out_v, out_hbm.at[pl.ds(wid * RPW * NFP, RPW * NFP)])


@jax.jit
def kernel(x, periods):
    pp = jnp.pad(periods, ((0, 0), (0, NFP - NF))).reshape(-1)
    xs = jnp.pad(x.reshape(BATCH, NF, FRAME), ((0, 0), (0, 0), (0, 1)))
    xs = jnp.pad(xs.reshape(BATCH, NF * FSTRIDE),
                 ((0, 0), (0, FROW - NF * FSTRIDE)))
    run = pl.kernel(
        _sc_body,
        out_type=jax.ShapeDtypeStruct((BATCH * NFP,), jnp.float32),
        mesh=plsc.VectorSubcoreMesh(core_axis_name="c", subcore_axis_name="s"),
        scratch_types=[
            pltpu.VMEM((XBUF,), jnp.float32),
            pltpu.VMEM((XBUF,), jnp.float32),
            pltpu.VMEM((FBUF,), jnp.float32),
            pltpu.VMEM((FBUF,), jnp.float32),
            pltpu.VMEM((RPW * NFP,), jnp.int32),
            pltpu.VMEM((RPW * NFP,), jnp.float32),
            pltpu.SemaphoreType.DMA,
            pltpu.SemaphoreType.DMA,
            pltpu.SemaphoreType.DMA,
            pltpu.SemaphoreType.DMA,
        ],
        compiler_params=pltpu.CompilerParams(needs_layout_passes=False),
    )
    out = run(x, xs, pp)
    return out.reshape(BATCH, NFP)[:, :NF].reshape(BATCH, 1, NF, 1)


# speedup vs baseline: 2.2546x; 2.2546x over previous
import jax
import jax.numpy as jnp
from jax import lax
from jax.experimental import pallas as pl
from jax.experimental.pallas import tpu as pltpu
from jax.experimental.pallas import tpu_sc as plsc

FRAME = 80
PMAX = 300
BATCH = 128
NF = 1000
NS = FRAME * NF
LANES = 16
NFP = 1024
NWORKERS = 32
RPW = BATCH // NWORKERS

HALO = 384
F0 = (0, 256, 512, 768)
NT = (256, 256, 256, 232)
NG = (16, 16, 16, 15)
SRC = (0, 80 * 256 - HALO, 80 * 512 - HALO, 80 * 768 - HALO)
LEN = (80 * 256, 80 * 512 - SRC[1], 80 * 768 - SRC[2], NS - SRC[3])
DST = (HALO, 0, 0, 0)
XBUF = HALO + LEN[1]


def _rsqrt(v):
    i = plsc.bitcast(v, jnp.int32)
    i = 0x5F3759DF - lax.shift_right_arithmetic(i, 1)
    y = plsc.bitcast(i, jnp.float32)
    for _ in range(4):
        y = y * (1.5 - 0.5 * v * y * y)
    return y


def _sc_body(x_hbm, per_hbm, out_hbm, xb0, xb1, pv, out_v, sem0, sem1):
    cid = lax.axis_index("c")
    sid = lax.axis_index("s")
    wid = sid * 2 + cid
    iota = lax.iota(jnp.int32, LANES)
    xbufs = (xb0, xb1)
    sems = (sem0, sem1)
    zero = jnp.zeros((LANES,), jnp.float32)

    pltpu.sync_copy(per_hbm.at[pl.ds(wid * RPW * NFP, RPW * NFP)], pv)

    def copy_refs(q, b):
        buf = xbufs[q % 2]
        return (x_hbm.at[b, 0, pl.ds(SRC[q], LEN[q])],
                buf.at[pl.ds(DST[q], LEN[q])])

    def start(q, b):
        src, dst = copy_refs(q, b)
        pltpu.async_copy(src, dst, sems[q % 2])

    def wait(q, b):
        src, dst = copy_refs(q, b)
        pltpu.make_async_copy(src, dst, sems[q % 2]).wait()

    def compute(q, r, b):
        buf = xbufs[q % 2]
        if q == 0:
            for z in range(HALO // LANES):
                buf[pl.ds(z * LANES, LANES)] = zero

        pbase = r * NFP + F0[q]

        @plsc.parallel_loop(0, NG[q], step=1, carry=jnp.int32(0))
        def group(g, carry):
            t = g * LANES + iota
            if q == 3:
                t = jnp.minimum(t, NT[3] - 1)
            p = plsc.load_gather(pv, [pbase + t])
            fb = t * FRAME + HALO
            lb = fb - p
            acc = (zero,) * 12 + (fb, lb)

            def body4(k, acc):
                d0, d1, d2, d3, e0, e1, e2, e3, l0, l1, l2, l3, bi, li = acc
                fa = plsc.load_gather(buf, [bi])
                la = plsc.load_gather(buf, [li])
                fb_ = plsc.load_gather(buf, [bi + 1])
                lb_ = plsc.load_gather(buf, [li + 1])
                fc = plsc.load_gather(buf, [bi + 2])
                lc = plsc.load_gather(buf, [li + 2])
                fd = plsc.load_gather(buf, [bi + 3])
                ld = plsc.load_gather(buf, [li + 3])
                return (d0 + fa * la, d1 + fb_ * lb_, d2 + fc * lc, d3 + fd * ld,
                        e0 + fa * fa, e1 + fb_ * fb_, e2 + fc * fc, e3 + fd * fd,
                        l0 + la * la, l1 + lb_ * lb_, l2 + lc * lc, l3 + ld * ld,
                        bi + 4, li + 4)

            acc = lax.fori_loop(0, FRAME // 4, body4, acc, unroll=5)
            d = (acc[0] + acc[1]) + (acc[2] + acc[3])
            fe = (acc[4] + acc[5]) + (acc[6] + acc[7])
            le = (acc[8] + acc[9]) + (acc[10] + acc[11])
            res = d * _rsqrt(fe * le + 1e-9)
            out_v[pl.ds(pbase + g * LANES, LANES)] = res
            return carry

    def row(r, carry):
        b = wid * RPW + r
        bnext = wid * RPW + jnp.minimum(r + 1, RPW - 1)
        start(1, b)
        wait(0, b)
        compute(0, r, b)
        start(2, b)
        wait(1, b)
        compute(1, r, b)
        start(3, b)
        wait(2, b)
        compute(2, r, b)
        start(0, bnext)
        wait(3, b)
        compute(3, r, b)
        return carry

    start(0, wid * RPW)
    lax.fori_loop(0, RPW, row, 0)
    wait(0, wid * RPW + RPW - 1)

    pltpu.sync_copy(out_v, out_hbm.at[pl.ds(wid * RPW * NFP, RPW * NFP)])


@jax.jit
def kernel(x, periods):
    pp = jnp.pad(periods, ((0, 0), (0, NFP - NF))).reshape(-1)
    run = pl.kernel(
        _sc_body,
        out_type=jax.ShapeDtypeStruct((BATCH * NFP,), jnp.float32),
        mesh=plsc.VectorSubcoreMesh(core_axis_name="c", subcore_axis_name="s"),
        scratch_types=[
            pltpu.VMEM((XBUF,), jnp.float32),
            pltpu.VMEM((XBUF,), jnp.float32),
            pltpu.VMEM((RPW * NFP,), jnp.int32),
            pltpu.VMEM((RPW * NFP,), jnp.float32),
            pltpu.SemaphoreType.DMA,
            pltpu.SemaphoreType.DMA,
        ],
        compiler_params=pltpu.CompilerParams(needs_layout_passes=False),
    )
    out = run(x, pp)
    return out.reshape(BATCH, NFP)[:, :NF].reshape(BATCH, 1, NF, 1)


# speedup vs baseline: 3.1228x; 1.3850x over previous
import jax
import jax.numpy as jnp
from jax import lax
from jax.experimental import pallas as pl
from jax.experimental.pallas import tpu as pltpu
from jax.experimental.pallas import tpu_sc as plsc

FRAME = 80
PMAX = 300
BATCH = 128
NF = 1000
NS = FRAME * NF
LANES = 16
NFP = 1024
NWORKERS = 32
RPW = BATCH // NWORKERS

HALO = 384
F0 = (0, 256, 512, 768)
NT = (256, 256, 256, 232)
NG = (16, 16, 16, 15)
SRC = (0, 80 * 256 - HALO, 80 * 512 - HALO, 80 * 768 - HALO)
LEN = (80 * 256, 80 * 512 - SRC[1], 80 * 768 - SRC[2], NS - SRC[3])
DST = (HALO, 0, 0, 0)
XBUF = HALO + LEN[1]


def _rsqrt(v):
    i = plsc.bitcast(v, jnp.int32)
    i = 0x5F3759DF - lax.shift_right_arithmetic(i, 1)
    y = plsc.bitcast(i, jnp.float32)
    for _ in range(4):
        y = y * (1.5 - 0.5 * v * y * y)
    return y


def _sc_body(x_hbm, per_hbm, out_hbm, xb0, xb1, pv, out_v, wtab, sem0, sem1):
    cid = lax.axis_index("c")
    sid = lax.axis_index("s")
    wid = sid * 2 + cid
    iota = lax.iota(jnp.int32, LANES)
    xbufs = (xb0, xb1)
    sems = (sem0, sem1)
    zero = jnp.zeros((LANES,), jnp.float32)

    pltpu.sync_copy(per_hbm.at[pl.ds(wid * RPW * NFP, RPW * NFP)], pv)

    zi = jnp.zeros((LANES,), jnp.int32)
    for z in range(LANES):
        wtab[pl.ds(z * LANES, LANES)] = zi
    plsc.store_scatter(wtab, [iota * 15 + LANES], zi + FRAME, mask=iota >= 1)

    def copy_refs(q, b):
        buf = xbufs[q % 2]
        return (x_hbm.at[b, 0, pl.ds(SRC[q], LEN[q])],
                buf.at[pl.ds(DST[q], LEN[q])])

    def start(q, b):
        src, dst = copy_refs(q, b)
        pltpu.async_copy(src, dst, sems[q % 2])

    def wait(q, b):
        src, dst = copy_refs(q, b)
        pltpu.make_async_copy(src, dst, sems[q % 2]).wait()

    def compute(q, r, b):
        buf = xbufs[q % 2]
        if q == 0:
            for z in range(HALO // LANES):
                buf[pl.ds(z * LANES, LANES)] = zero

        pbase = r * NFP + F0[q]

        @plsc.parallel_loop(0, NG[q], step=1, carry=jnp.int32(0))
        def group(g, carry):
            t = g * LANES + iota
            if q == 3:
                t = jnp.minimum(t, NT[3] - 1)
            p = plsc.load_gather(pv, [pbase + t])
            fb = t * FRAME + HALO + iota * 5
            acc = (zero,) * 12 + (fb,)

            def body5(k, acc):
                d0, d1, d2, d3, e0, e1, e2, e3, g0, g1, g2, g3, fi = acc
                w = wtab[pl.ds(k * LANES, LANES)]
                f0 = fi - w
                l0 = f0 - p
                fv0 = plsc.load_gather(buf, [f0])
                lv0 = plsc.load_gather(buf, [l0])
                fv1 = plsc.load_gather(buf, [f0 + 1])
                lv1 = plsc.load_gather(buf, [l0 + 1])
                fv2 = plsc.load_gather(buf, [f0 + 2])
                lv2 = plsc.load_gather(buf, [l0 + 2])
                fv3 = plsc.load_gather(buf, [f0 + 3])
                lv3 = plsc.load_gather(buf, [l0 + 3])
                fv4 = plsc.load_gather(buf, [f0 + 4])
                lv4 = plsc.load_gather(buf, [l0 + 4])
                return ((d0 + fv0 * lv0) + fv4 * lv4, d1 + fv1 * lv1,
                        d2 + fv2 * lv2, d3 + fv3 * lv3,
                        (e0 + fv0 * fv0) + fv4 * fv4, e1 + fv1 * fv1,
                        e2 + fv2 * fv2, e3 + fv3 * fv3,
                        (g0 + lv0 * lv0) + lv4 * lv4, g1 + lv1 * lv1,
                        g2 + lv2 * lv2, g3 + lv3 * lv3,
                        f0 + 5)

            acc = lax.fori_loop(0, FRAME // 5, body5, acc, unroll=4)
            d = (acc[0] + acc[1]) + (acc[2] + acc[3])
            fe = (acc[4] + acc[5]) + (acc[6] + acc[7])
            le = (acc[8] + acc[9]) + (acc[10] + acc[11])
            res = d * _rsqrt(fe * le + 1e-9)
            out_v[pl.ds(pbase + g * LANES, LANES)] = res
            return carry

    def row(r, carry):
        b = wid * RPW + r
        bnext = wid * RPW + jnp.minimum(r + 1, RPW - 1)
        start(1, b)
        wait(0, b)
        compute(0, r, b)
        start(2, b)
        wait(1, b)
        compute(1, r, b)
        start(3, b)
        wait(2, b)
        compute(2, r, b)
        start(0, bnext)
        wait(3, b)
        compute(3, r, b)
        return carry

    start(0, wid * RPW)
    lax.fori_loop(0, RPW, row, 0)
    wait(0, wid * RPW + RPW - 1)

    pltpu.sync_copy(out_v, out_hbm.at[pl.ds(wid * RPW * NFP, RPW * NFP)])


@jax.jit
def kernel(x, periods):
    pp = jnp.pad(periods, ((0, 0), (0, NFP - NF))).reshape(-1)
    run = pl.kernel(
        _sc_body,
        out_type=jax.ShapeDtypeStruct((BATCH * NFP,), jnp.float32),
        mesh=plsc.VectorSubcoreMesh(core_axis_name="c", subcore_axis_name="s"),
        scratch_types=[
            pltpu.VMEM((XBUF,), jnp.float32),
            pltpu.VMEM((XBUF,), jnp.float32),
            pltpu.VMEM((RPW * NFP,), jnp.int32),
            pltpu.VMEM((RPW * NFP,), jnp.float32),
            pltpu.VMEM((LANES * LANES,), jnp.int32),
            pltpu.SemaphoreType.DMA,
            pltpu.SemaphoreType.DMA,
        ],
        compiler_params=pltpu.CompilerParams(needs_layout_passes=False),
    )
    out = run(x, pp)
    return out.reshape(BATCH, NFP)[:, :NF].reshape(BATCH, 1, NF, 1)
